# baseline (speedup 1.0000x reference)
import functools

import jax
import jax.numpy as jnp
from jax import lax
from jax.experimental import pallas as pl
from jax.experimental.pallas import tpu as pltpu
from jax.experimental.pallas import tpu_sc as plsc

DIM = 64
N_EMBED = 1024
POS_DIM = 16
POS_EMBED = 128
TOKENS = 16384
BLOCK = 1024
GRID = TOKENS // BLOCK


def _tc_body(x_ref, emb_ref, gate_ref, ind_ref, eind_ref, diff_ref, cb_ref):
    b = pl.program_id(0)
    x = x_ref[...]
    emb = emb_ref[...]

    row = lax.broadcasted_iota(jnp.int32, (DIM, N_EMBED), 0)
    col = lax.broadcasted_iota(jnp.int32, (DIM, N_EMBED), 1)
    mask = ((row < DIM - POS_DIM) == (col < N_EMBED - POS_EMBED)).astype(
        jnp.float32)
    emb_masked = emb * mask
    cb = emb_masked * gate_ref[0, 0]

    @pl.when(b == 0)
    def _():
        cb_ref[...] = emb_masked

    xcb = jnp.dot(x, cb, preferred_element_type=jnp.float32)
    cnorm = jnp.sum(cb * cb, axis=0, keepdims=True)
    s = cnorm - 2.0 * xcb
    mins = jnp.min(s, axis=1, keepdims=True)
    onehot = (s <= mins).astype(jnp.float32)
    r2 = lax.broadcasted_iota(jnp.int32, (2, N_EMBED), 0)
    j2 = lax.broadcasted_iota(jnp.int32, (2, N_EMBED), 1)
    w2 = jnp.where(r2 == 0, j2 // 8, j2 % 8).astype(jnp.float32)
    hl = lax.dot_general(w2, onehot, (((1,), (1,)), ((), ())),
                         preferred_element_type=jnp.float32)
    ind = 8.0 * lax.slice(hl, (0, 0), (1, BLOCK)) + lax.slice(
        hl, (1, 0), (2, BLOCK))
    ind = jnp.minimum(ind, jnp.float32(N_EMBED - 1))
    ind_row = ind.astype(jnp.int32)
    ind_ref[...] = ind_row.reshape(1, 1, N_EMBED)
    eind_ref[...] = ind_row.reshape(1, 32, 32)

    fnorm = jnp.sum(x * x, axis=1, keepdims=True)
    diff_ref[...] = ((jnp.sum(fnorm) + jnp.sum(mins))
                     / jnp.float32(BLOCK * DIM)).reshape(1, 1, 1)


def _tc_stage(flat, embed, gate):
    return pl.pallas_call(
        _tc_body,
        grid=(GRID,),
        in_specs=[
            pl.BlockSpec((BLOCK, DIM), lambda b: (b, 0)),
            pl.BlockSpec((DIM, N_EMBED), lambda b: (0, 0)),
            pl.BlockSpec((1, 1), lambda b: (0, 0)),
        ],
        out_specs=[
            pl.BlockSpec((1, 1, N_EMBED), lambda b: (b, 0, 0)),
            pl.BlockSpec((1, 32, 32), lambda b: (b, 0, 0)),
            pl.BlockSpec((1, 1, 1), lambda b: (b, 0, 0)),
            pl.BlockSpec((DIM, N_EMBED), lambda b: (0, 0)),
        ],
        out_shape=[
            jax.ShapeDtypeStruct((GRID, 1, N_EMBED), jnp.int32),
            jax.ShapeDtypeStruct((GRID, 32, 32), jnp.int32),
            jax.ShapeDtypeStruct((GRID, 1, 1), jnp.float32),
            jax.ShapeDtypeStruct((DIM, N_EMBED), jnp.float32),
        ],
    )(flat, embed, gate)


_CHUNK = 128


def _sc_gather(table, idx2d):
    info = plsc.get_sparse_core_info()
    _NC, _NS = info.num_cores, info.num_subcores
    _NW = _NC * _NS
    _BPW = TOKENS // _NW
    _NCHUNK = _BPW // _CHUNK
    mesh = plsc.VectorSubcoreMesh(core_axis_name="c", subcore_axis_name="s")

    @functools.partial(
        pl.kernel,
        mesh=mesh,
        compiler_params=pltpu.CompilerParams(use_tc_tiling_on_sc=False),
        out_type=jax.ShapeDtypeStruct((TOKENS, DIM), jnp.float32),
        scratch_types=[
            pltpu.VMEM((_NCHUNK, _CHUNK), jnp.int32),
            pltpu.VMEM((_BPW, DIM), jnp.float32),
            pltpu.SemaphoreType.DMA,
        ],
    )
    def k(table_hbm, idx_hbm, out_hbm, idx_v, rows_v, sem):
        wid = lax.axis_index("s") * _NC + lax.axis_index("c")
        pltpu.sync_copy(idx_hbm.at[pl.ds(wid * _NCHUNK, _NCHUNK), :], idx_v)
        copies = [
            pltpu.async_copy(
                table_hbm.at[idx_v.at[j]],
                rows_v.at[pl.ds(j * _CHUNK, _CHUNK), :],
                sem,
            )
            for j in range(_NCHUNK)
        ]
        for c in copies:
            c.wait()
        pltpu.sync_copy(rows_v, out_hbm.at[pl.ds(wid * _BPW, _BPW), :])

    return k(table, idx2d)


def kernel(input, embed, bi):
    flat = input.reshape(TOKENS, DIM)
    gate = (jnp.asarray(bi) == 1).astype(jnp.float32).reshape(1, 1)
    ind, embed_ind, diff, cb = _tc_stage(flat, embed, gate)
    q = _sc_gather(embed.T, ind.reshape(TOKENS // _CHUNK, _CHUNK))
    ind_flat = ind.reshape(TOKENS)
    keep = ((lax.broadcasted_iota(jnp.int32, (TOKENS, DIM), 1)
             < DIM - POS_DIM)
            == (ind_flat < N_EMBED - POS_EMBED)[:, None])
    quantize = (jnp.where(keep, q, 0.0) * gate[0, 0]).reshape(input.shape)
    return quantize, diff.reshape(GRID), embed_ind, cb


# speedup vs baseline: 1.0169x; 1.0169x over previous
import functools

import jax
import jax.numpy as jnp
from jax import lax
from jax.experimental import pallas as pl
from jax.experimental.pallas import tpu as pltpu
from jax.experimental.pallas import tpu_sc as plsc

DIM = 64
N_EMBED = 1024
POS_DIM = 16
POS_EMBED = 128
TOKENS = 16384
BLOCK = 1024
GRID = TOKENS // BLOCK


def _tc_body(x_ref, emb_ref, gate_ref, ind_ref, eind_ref, diff_ref, cb_ref):
    b = pl.program_id(0)
    x = x_ref[...]
    emb = emb_ref[...]

    row = lax.broadcasted_iota(jnp.int32, (DIM, N_EMBED), 0)
    col = lax.broadcasted_iota(jnp.int32, (DIM, N_EMBED), 1)
    mask = ((row < DIM - POS_DIM) == (col < N_EMBED - POS_EMBED)).astype(
        jnp.float32)
    emb_masked = emb * mask
    cb = emb_masked * gate_ref[0, 0]

    @pl.when(b == 0)
    def _():
        cb_ref[...] = emb_masked

    xcb = jnp.dot(x, cb, preferred_element_type=jnp.float32)
    cnorm = jnp.sum(cb * cb, axis=0, keepdims=True)
    s = cnorm - 2.0 * xcb
    mins = jnp.min(s, axis=1, keepdims=True)
    onehot = (s <= mins).astype(jnp.float32)
    r2 = lax.broadcasted_iota(jnp.int32, (2, N_EMBED), 0)
    j2 = lax.broadcasted_iota(jnp.int32, (2, N_EMBED), 1)
    w2 = jnp.where(r2 == 0, j2 // 8, j2 % 8).astype(jnp.float32)
    hl = lax.dot_general(w2, onehot, (((1,), (1,)), ((), ())),
                         preferred_element_type=jnp.float32)
    ind = 8.0 * lax.slice(hl, (0, 0), (1, BLOCK)) + lax.slice(
        hl, (1, 0), (2, BLOCK))
    ind = jnp.minimum(ind, jnp.float32(N_EMBED - 1))
    ind_row = ind.astype(jnp.int32)
    ind_ref[...] = ind_row.reshape(1, 1, N_EMBED)
    eind_ref[...] = ind_row.reshape(1, 32, 32)

    fnorm = jnp.sum(x * x, axis=1, keepdims=True)
    diff_ref[...] = ((jnp.sum(fnorm) + jnp.sum(mins))
                     / jnp.float32(BLOCK * DIM)).reshape(1, 1, 1)


def _tc_stage(flat, embed, gate, grid):
    return pl.pallas_call(
        _tc_body,
        grid=(grid,),
        in_specs=[
            pl.BlockSpec((BLOCK, DIM), lambda b: (b, 0)),
            pl.BlockSpec((DIM, N_EMBED), lambda b: (0, 0)),
            pl.BlockSpec((1, 1), lambda b: (0, 0)),
        ],
        out_specs=[
            pl.BlockSpec((1, 1, N_EMBED), lambda b: (b, 0, 0)),
            pl.BlockSpec((1, 32, 32), lambda b: (b, 0, 0)),
            pl.BlockSpec((1, 1, 1), lambda b: (b, 0, 0)),
            pl.BlockSpec((DIM, N_EMBED), lambda b: (0, 0)),
        ],
        out_shape=[
            jax.ShapeDtypeStruct((grid, 1, N_EMBED), jnp.int32),
            jax.ShapeDtypeStruct((grid, 32, 32), jnp.int32),
            jax.ShapeDtypeStruct((grid, 1, 1), jnp.float32),
            jax.ShapeDtypeStruct((DIM, N_EMBED), jnp.float32),
        ],
    )(flat, embed, gate)


_CHUNK = 128


def _sc_gather(table, idx2d, ntok):
    info = plsc.get_sparse_core_info()
    _NC, _NS = info.num_cores, info.num_subcores
    _NW = _NC * _NS
    _BPW = ntok // _NW
    _NCHUNK = _BPW // _CHUNK
    mesh = plsc.VectorSubcoreMesh(core_axis_name="c", subcore_axis_name="s")

    @functools.partial(
        pl.kernel,
        mesh=mesh,
        compiler_params=pltpu.CompilerParams(use_tc_tiling_on_sc=False),
        out_type=jax.ShapeDtypeStruct((ntok, DIM), jnp.float32),
        scratch_types=[
            pltpu.VMEM((_NCHUNK, _CHUNK), jnp.int32),
            pltpu.VMEM((_BPW, DIM), jnp.float32),
            pltpu.SemaphoreType.DMA,
            pltpu.SemaphoreType.DMA,
        ],
    )
    def k(table_hbm, idx_hbm, out_hbm, idx_v, rows_v, gsem, wsem):
        wid = lax.axis_index("s") * _NC + lax.axis_index("c")
        pltpu.sync_copy(idx_hbm.at[pl.ds(wid * _NCHUNK, _NCHUNK), :], idx_v)
        gathers = [
            pltpu.async_copy(
                table_hbm.at[idx_v.at[j]],
                rows_v.at[pl.ds(j * _CHUNK, _CHUNK), :],
                gsem,
            )
            for j in range(_NCHUNK)
        ]
        writes = []
        for j in range(_NCHUNK):
            gathers[j].wait()
            writes.append(pltpu.async_copy(
                rows_v.at[pl.ds(j * _CHUNK, _CHUNK), :],
                out_hbm.at[pl.ds(wid * _BPW + j * _CHUNK, _CHUNK), :],
                wsem,
            ))
        for w in writes:
            w.wait()

    return k(table, idx2d)


def _half(input_half, embed, gate, table):
    ntok = TOKENS // 2
    flat = input_half.reshape(ntok, DIM)
    ind, eind, diff, cb = _tc_stage(flat, embed, gate, GRID // 2)
    q = _sc_gather(table, ind.reshape(ntok // _CHUNK, _CHUNK), ntok)
    ind_flat = ind.reshape(ntok)
    keep = ((lax.broadcasted_iota(jnp.int32, (ntok, DIM), 1)
             < DIM - POS_DIM)
            == (ind_flat < N_EMBED - POS_EMBED)[:, None])
    quantize = jnp.where(keep, q, 0.0).reshape(input_half.shape)
    return quantize, eind, diff, cb


def kernel(input, embed, bi):
    gate = (jnp.asarray(bi) == 1).astype(jnp.float32).reshape(1, 1)
    table = embed.T
    qa, ea, da, cb = _half(input[:GRID // 2], embed, gate, table)
    qb, eb, db, _ = _half(input[GRID // 2:], embed, gate, table)
    quantize = jnp.concatenate([qa, qb], axis=0) * gate[0, 0]
    diff = jnp.concatenate([da, db], axis=0).reshape(GRID)
    embed_ind = jnp.concatenate([ea, eb], axis=0)
    return quantize, diff, embed_ind, cb


# speedup vs baseline: 1.1051x; 1.0867x over previous
import functools

import jax
import jax.numpy as jnp
from jax import lax
from jax.experimental import pallas as pl
from jax.experimental.pallas import tpu as pltpu
from jax.experimental.pallas import tpu_sc as plsc

DIM = 64
N_EMBED = 1024
POS_DIM = 16
POS_EMBED = 128
TOKENS = 16384
BLOCK = 1024
GRID = TOKENS // BLOCK


def _tc_body(x_ref, emb_ref, gate_ref, ind_ref, eind_ref, diff_ref, cb_ref):
    b = pl.program_id(0)
    x = x_ref[...]
    emb = emb_ref[...]

    row = lax.broadcasted_iota(jnp.int32, (DIM, N_EMBED), 0)
    col = lax.broadcasted_iota(jnp.int32, (DIM, N_EMBED), 1)
    mask = ((row < DIM - POS_DIM) == (col < N_EMBED - POS_EMBED)).astype(
        jnp.float32)
    emb_masked = emb * mask
    cb = emb_masked * gate_ref[0, 0]

    @pl.when(b == 0)
    def _():
        cb_ref[...] = emb_masked

    xcb = jnp.dot(x, cb, preferred_element_type=jnp.float32)
    cnorm = jnp.sum(cb * cb, axis=0, keepdims=True)
    s = cnorm - 2.0 * xcb
    mins = jnp.min(s, axis=1, keepdims=True)
    onehot = (s <= mins).astype(jnp.float32)
    r2 = lax.broadcasted_iota(jnp.int32, (2, N_EMBED), 0)
    j2 = lax.broadcasted_iota(jnp.int32, (2, N_EMBED), 1)
    w2 = jnp.where(r2 == 0, j2 // 8, j2 % 8).astype(jnp.float32)
    hl = lax.dot_general(w2, onehot, (((1,), (1,)), ((), ())),
                         preferred_element_type=jnp.float32)
    ind = 8.0 * lax.slice(hl, (0, 0), (1, BLOCK)) + lax.slice(
        hl, (1, 0), (2, BLOCK))
    ind = jnp.minimum(ind, jnp.float32(N_EMBED - 1))
    ind_row = ind.astype(jnp.int32)
    ind_ref[...] = ind_row.reshape(1, 1, N_EMBED)
    eind_ref[...] = ind_row.reshape(1, 32, 32)

    fnorm = jnp.sum(x * x, axis=1, keepdims=True)
    diff_ref[...] = ((jnp.sum(fnorm) + jnp.sum(mins))
                     / jnp.float32(BLOCK * DIM)).reshape(1, 1, 1)


def _tc_stage(flat, embed, gate, grid, base):
    return pl.pallas_call(
        _tc_body,
        grid=(grid,),
        in_specs=[
            pl.BlockSpec((BLOCK, DIM), lambda b: (b + base, 0)),
            pl.BlockSpec((DIM, N_EMBED), lambda b: (0, 0)),
            pl.BlockSpec((1, 1), lambda b: (0, 0)),
        ],
        out_specs=[
            pl.BlockSpec((1, 1, N_EMBED), lambda b: (b, 0, 0)),
            pl.BlockSpec((1, 32, 32), lambda b: (b, 0, 0)),
            pl.BlockSpec((1, 1, 1), lambda b: (b, 0, 0)),
            pl.BlockSpec((DIM, N_EMBED), lambda b: (0, 0)),
        ],
        out_shape=[
            jax.ShapeDtypeStruct((grid, 1, N_EMBED), jnp.int32),
            jax.ShapeDtypeStruct((grid, 32, 32), jnp.int32),
            jax.ShapeDtypeStruct((grid, 1, 1), jnp.float32),
            jax.ShapeDtypeStruct((DIM, N_EMBED), jnp.float32),
        ],
    )(flat, embed, gate)


_CHUNK = 128


def _sc_gather(table, idx2d, ntok):
    info = plsc.get_sparse_core_info()
    _NC, _NS = info.num_cores, info.num_subcores
    _NW = _NC * _NS
    _BPW = ntok // _NW
    _NCHUNK = _BPW // _CHUNK
    mesh = plsc.VectorSubcoreMesh(core_axis_name="c", subcore_axis_name="s")

    @functools.partial(
        pl.kernel,
        mesh=mesh,
        compiler_params=pltpu.CompilerParams(use_tc_tiling_on_sc=False),
        out_type=jax.ShapeDtypeStruct((ntok, DIM), jnp.float32),
        scratch_types=[
            pltpu.VMEM((_NCHUNK, _CHUNK), jnp.int32),
            pltpu.VMEM((_BPW, DIM), jnp.float32),
            pltpu.SemaphoreType.DMA,
            pltpu.SemaphoreType.DMA,
        ],
    )
    def k(table_hbm, idx_hbm, out_hbm, idx_v, rows_v, gsem, wsem):
        wid = lax.axis_index("s") * _NC + lax.axis_index("c")
        pltpu.sync_copy(idx_hbm.at[pl.ds(wid * _NCHUNK, _NCHUNK), :], idx_v)
        gathers = [
            pltpu.async_copy(
                table_hbm.at[idx_v.at[j]],
                rows_v.at[pl.ds(j * _CHUNK, _CHUNK), :],
                gsem,
            )
            for j in range(_NCHUNK)
        ]
        writes = []
        for j in range(_NCHUNK):
            gathers[j].wait()
            writes.append(pltpu.async_copy(
                rows_v.at[pl.ds(j * _CHUNK, _CHUNK), :],
                out_hbm.at[pl.ds(wid * _BPW + j * _CHUNK, _CHUNK), :],
                wsem,
            ))
        for w in writes:
            w.wait()

    return k(table, idx2d)


def kernel(input, embed, bi):
    gate = (jnp.asarray(bi) == 1).astype(jnp.float32).reshape(1, 1)
    table = embed.T
    flat = input.reshape(TOKENS, DIM)
    htok = TOKENS // 2
    inda, ea, da, cb = _tc_stage(flat, embed, gate, GRID // 2, 0)
    qa = _sc_gather(table, inda.reshape(htok // _CHUNK, _CHUNK), htok)
    indb, eb, db, _ = _tc_stage(flat, embed, gate, GRID // 2, GRID // 2)
    qb = _sc_gather(table, indb.reshape(htok // _CHUNK, _CHUNK), htok)
    q = jnp.concatenate([qa, qb], axis=0)
    ind_flat = jnp.concatenate([inda.reshape(htok), indb.reshape(htok)])
    keep = ((lax.broadcasted_iota(jnp.int32, (TOKENS, DIM), 1)
             < DIM - POS_DIM)
            == (ind_flat < N_EMBED - POS_EMBED)[:, None])
    quantize = (jnp.where(keep, q, 0.0) * gate[0, 0]).reshape(input.shape)
    diff = jnp.concatenate([da, db], axis=0).reshape(GRID)
    embed_ind = jnp.concatenate([ea, eb], axis=0)
    return quantize, diff, embed_ind, cb


# speedup vs baseline: 1.1843x; 1.0716x over previous
import functools

import jax
import jax.numpy as jnp
from jax import lax
from jax.experimental import pallas as pl
from jax.experimental.pallas import tpu as pltpu
from jax.experimental.pallas import tpu_sc as plsc

DIM = 64
N_EMBED = 1024
POS_DIM = 16
POS_EMBED = 128
TOKENS = 16384
BLOCK = 1024
GRID = TOKENS // BLOCK


def _tc_common(x, emb, gate):
    row = lax.broadcasted_iota(jnp.int32, (DIM, N_EMBED), 0)
    col = lax.broadcasted_iota(jnp.int32, (DIM, N_EMBED), 1)
    mask = ((row < DIM - POS_DIM) == (col < N_EMBED - POS_EMBED)).astype(
        jnp.float32)
    emb_masked = emb * mask
    cb = emb_masked * gate

    xcb = jnp.dot(x, cb, preferred_element_type=jnp.float32)
    cnorm = jnp.sum(cb * cb, axis=0, keepdims=True)
    s = cnorm - 2.0 * xcb
    mins = jnp.min(s, axis=1, keepdims=True)
    onehot = (s <= mins).astype(jnp.float32)
    r2 = lax.broadcasted_iota(jnp.int32, (2, N_EMBED), 0)
    j2 = lax.broadcasted_iota(jnp.int32, (2, N_EMBED), 1)
    w2 = jnp.where(r2 == 0, j2 // 8, j2 % 8).astype(jnp.float32)
    hl = lax.dot_general(w2, onehot, (((1,), (1,)), ((), ())),
                         preferred_element_type=jnp.float32)
    ind = 8.0 * lax.slice(hl, (0, 0), (1, BLOCK)) + lax.slice(
        hl, (1, 0), (2, BLOCK))
    ind = jnp.minimum(ind, jnp.float32(N_EMBED - 1))
    ind_row = ind.astype(jnp.int32)

    fnorm = jnp.sum(x * x, axis=1, keepdims=True)
    diff = ((jnp.sum(fnorm) + jnp.sum(mins)) / jnp.float32(BLOCK * DIM))
    return emb_masked, cb, onehot, ind_row, diff


def _tc_body_a(x_ref, emb_ref, gate_ref, ind_ref, eind_ref, diff_ref, cb_ref):
    b = pl.program_id(0)
    emb_masked, _, _, ind_row, diff = _tc_common(
        x_ref[...], emb_ref[...], gate_ref[0, 0])

    @pl.when(b == 0)
    def _():
        cb_ref[...] = emb_masked

    ind_ref[...] = ind_row.reshape(1, 1, N_EMBED)
    eind_ref[...] = ind_row.reshape(1, 32, 32)
    diff_ref[...] = diff.reshape(1, 1, 1)


def _tc_body_b(x_ref, emb_ref, gate_ref, ind_ref, eind_ref, diff_ref, q_ref):
    _, cb, onehot, ind_row, diff = _tc_common(
        x_ref[...], emb_ref[...], gate_ref[0, 0])
    ind_ref[...] = ind_row.reshape(1, 1, N_EMBED)
    eind_ref[...] = ind_row.reshape(1, 32, 32)
    diff_ref[...] = diff.reshape(1, 1, 1)
    cb_hi = cb.astype(jnp.bfloat16).astype(jnp.float32)
    cb_lo = cb - cb_hi
    nt = (((1,), (1,)), ((), ()))
    q = (lax.dot_general(onehot, cb_hi, nt, preferred_element_type=jnp.float32)
         + lax.dot_general(onehot, cb_lo, nt,
                           preferred_element_type=jnp.float32))
    q_ref[...] = q


def _tc_stage(flat, embed, gate, grid, base, with_gather):
    last = [
        pl.BlockSpec((BLOCK, DIM), lambda b: (b, 0)),
        jax.ShapeDtypeStruct((grid * BLOCK, DIM), jnp.float32),
    ] if with_gather else [
        pl.BlockSpec((DIM, N_EMBED), lambda b: (0, 0)),
        jax.ShapeDtypeStruct((DIM, N_EMBED), jnp.float32),
    ]
    return pl.pallas_call(
        _tc_body_b if with_gather else _tc_body_a,
        grid=(grid,),
        in_specs=[
            pl.BlockSpec((BLOCK, DIM), lambda b: (b + base, 0)),
            pl.BlockSpec((DIM, N_EMBED), lambda b: (0, 0)),
            pl.BlockSpec((1, 1), lambda b: (0, 0)),
        ],
        out_specs=[
            pl.BlockSpec((1, 1, N_EMBED), lambda b: (b, 0, 0)),
            pl.BlockSpec((1, 32, 32), lambda b: (b, 0, 0)),
            pl.BlockSpec((1, 1, 1), lambda b: (b, 0, 0)),
            last[0],
        ],
        out_shape=[
            jax.ShapeDtypeStruct((grid, 1, N_EMBED), jnp.int32),
            jax.ShapeDtypeStruct((grid, 32, 32), jnp.int32),
            jax.ShapeDtypeStruct((grid, 1, 1), jnp.float32),
            last[1],
        ],
    )(flat, embed, gate)


_CHUNK = 128


def _sc_gather(table, idx2d, ntok):
    info = plsc.get_sparse_core_info()
    _NC, _NS = info.num_cores, info.num_subcores
    _NW = _NC * _NS
    _BPW = ntok // _NW
    _NCHUNK = _BPW // _CHUNK
    mesh = plsc.VectorSubcoreMesh(core_axis_name="c", subcore_axis_name="s")

    @functools.partial(
        pl.kernel,
        mesh=mesh,
        compiler_params=pltpu.CompilerParams(use_tc_tiling_on_sc=False),
        out_type=jax.ShapeDtypeStruct((ntok, DIM), jnp.float32),
        scratch_types=[
            pltpu.VMEM((_NCHUNK, _CHUNK), jnp.int32),
            pltpu.VMEM((_BPW, DIM), jnp.float32),
            pltpu.SemaphoreType.DMA,
            pltpu.SemaphoreType.DMA,
        ],
    )
    def k(table_hbm, idx_hbm, out_hbm, idx_v, rows_v, gsem, wsem):
        wid = lax.axis_index("s") * _NC + lax.axis_index("c")
        pltpu.sync_copy(idx_hbm.at[pl.ds(wid * _NCHUNK, _NCHUNK), :], idx_v)
        gathers = [
            pltpu.async_copy(
                table_hbm.at[idx_v.at[j]],
                rows_v.at[pl.ds(j * _CHUNK, _CHUNK), :],
                gsem,
            )
            for j in range(_NCHUNK)
        ]
        writes = []
        for j in range(_NCHUNK):
            gathers[j].wait()
            writes.append(pltpu.async_copy(
                rows_v.at[pl.ds(j * _CHUNK, _CHUNK), :],
                out_hbm.at[pl.ds(wid * _BPW + j * _CHUNK, _CHUNK), :],
                wsem,
            ))
        for w in writes:
            w.wait()

    return k(table, idx2d)


def kernel(input, embed, bi):
    gate = (jnp.asarray(bi) == 1).astype(jnp.float32).reshape(1, 1)
    table = embed.T
    flat = input.reshape(TOKENS, DIM)
    htok = TOKENS // 2
    inda, ea, da, cb = _tc_stage(flat, embed, gate, GRID // 2, 0, False)
    qa = _sc_gather(table, inda.reshape(htok // _CHUNK, _CHUNK), htok)
    indb, eb, db, qb = _tc_stage(flat, embed, gate, GRID // 2, GRID // 2, True)
    ind_a = inda.reshape(htok)
    keep = ((lax.broadcasted_iota(jnp.int32, (htok, DIM), 1)
             < DIM - POS_DIM)
            == (ind_a < N_EMBED - POS_EMBED)[:, None])
    qa_m = jnp.where(keep, qa, 0.0) * gate[0, 0]
    quantize = jnp.concatenate([qa_m, qb], axis=0).reshape(input.shape)
    diff = jnp.concatenate([da, db], axis=0).reshape(GRID)
    embed_ind = jnp.concatenate([ea, eb], axis=0)
    return quantize, diff, embed_ind, cb


# speedup vs baseline: 1.2308x; 1.0393x over previous
import functools

import jax
import jax.numpy as jnp
from jax import lax
from jax.experimental import pallas as pl
from jax.experimental.pallas import tpu as pltpu
from jax.experimental.pallas import tpu_sc as plsc

DIM = 64
N_EMBED = 1024
POS_DIM = 16
POS_EMBED = 128
TOKENS = 16384
BLOCK = 1024
GRID = TOKENS // BLOCK


def _tc_common(x, emb, gate):
    row = lax.broadcasted_iota(jnp.int32, (DIM, N_EMBED), 0)
    col = lax.broadcasted_iota(jnp.int32, (DIM, N_EMBED), 1)
    mask = ((row < DIM - POS_DIM) == (col < N_EMBED - POS_EMBED)).astype(
        jnp.float32)
    emb_masked = emb * mask
    cb = emb_masked * gate

    xcb = jnp.dot(x, cb, preferred_element_type=jnp.float32)
    cnorm = jnp.sum(cb * cb, axis=0, keepdims=True)
    s = cnorm - 2.0 * xcb
    mins = jnp.min(s, axis=1, keepdims=True)
    onehot = (s <= mins).astype(jnp.float32)
    r2 = lax.broadcasted_iota(jnp.int32, (2, N_EMBED), 0)
    j2 = lax.broadcasted_iota(jnp.int32, (2, N_EMBED), 1)
    w2 = jnp.where(r2 == 0, j2 // 8, j2 % 8).astype(jnp.float32)
    hl = lax.dot_general(w2, onehot, (((1,), (1,)), ((), ())),
                         preferred_element_type=jnp.float32)
    ind = 8.0 * lax.slice(hl, (0, 0), (1, BLOCK)) + lax.slice(
        hl, (1, 0), (2, BLOCK))
    ind = jnp.minimum(ind, jnp.float32(N_EMBED - 1))
    ind_row = ind.astype(jnp.int32)

    fnorm = jnp.sum(x * x, axis=1, keepdims=True)
    diff = ((jnp.sum(fnorm) + jnp.sum(mins)) / jnp.float32(BLOCK * DIM))
    return emb_masked, cb, onehot, ind_row, diff


def _tc_body_a(x_ref, emb_ref, gate_ref, ind_ref, eind_ref, diff_ref, cb_ref):
    b = pl.program_id(0)
    emb_masked, _, _, ind_row, diff = _tc_common(
        x_ref[...], emb_ref[...], gate_ref[0, 0])

    @pl.when(b == 0)
    def _():
        cb_ref[...] = emb_masked

    ind_ref[...] = ind_row.reshape(1, 1, N_EMBED)
    eind_ref[...] = ind_row.reshape(1, 32, 32)
    diff_ref[...] = diff.reshape(1, 1, 1)


def _tc_body_b(x_ref, emb_ref, gate_ref, ind_ref, eind_ref, diff_ref, q_ref):
    _, cb, onehot, ind_row, diff = _tc_common(
        x_ref[...], emb_ref[...], gate_ref[0, 0])
    ind_ref[...] = ind_row.reshape(1, 1, N_EMBED)
    eind_ref[...] = ind_row.reshape(1, 32, 32)
    diff_ref[...] = diff.reshape(1, 1, 1)
    cb_hi = cb.astype(jnp.bfloat16).astype(jnp.float32)
    cb_lo = cb - cb_hi
    cb_hl = jnp.concatenate([cb_hi, cb_lo], axis=0)
    nt = (((1,), (1,)), ((), ()))
    q2 = lax.dot_general(onehot, cb_hl, nt,
                         preferred_element_type=jnp.float32)
    q_ref[...] = (lax.slice(q2, (0, 0), (BLOCK, DIM))
                  + lax.slice(q2, (0, DIM), (BLOCK, 2 * DIM)))


def _tc_stage(flat, embed, gate, grid, base, with_gather):
    last = [
        pl.BlockSpec((BLOCK, DIM), lambda b: (b, 0)),
        jax.ShapeDtypeStruct((grid * BLOCK, DIM), jnp.float32),
    ] if with_gather else [
        pl.BlockSpec((DIM, N_EMBED), lambda b: (0, 0)),
        jax.ShapeDtypeStruct((DIM, N_EMBED), jnp.float32),
    ]
    return pl.pallas_call(
        _tc_body_b if with_gather else _tc_body_a,
        grid=(grid,),
        in_specs=[
            pl.BlockSpec((BLOCK, DIM), lambda b: (b + base, 0)),
            pl.BlockSpec((DIM, N_EMBED), lambda b: (0, 0)),
            pl.BlockSpec((1, 1), lambda b: (0, 0)),
        ],
        out_specs=[
            pl.BlockSpec((1, 1, N_EMBED), lambda b: (b, 0, 0)),
            pl.BlockSpec((1, 32, 32), lambda b: (b, 0, 0)),
            pl.BlockSpec((1, 1, 1), lambda b: (b, 0, 0)),
            last[0],
        ],
        out_shape=[
            jax.ShapeDtypeStruct((grid, 1, N_EMBED), jnp.int32),
            jax.ShapeDtypeStruct((grid, 32, 32), jnp.int32),
            jax.ShapeDtypeStruct((grid, 1, 1), jnp.float32),
            last[1],
        ],
    )(flat, embed, gate)


_CHUNK = 128


def _sc_gather(table, idx2d, ntok):
    info = plsc.get_sparse_core_info()
    _NC, _NS = info.num_cores, info.num_subcores
    _NW = _NC * _NS
    _BPW = ntok // _NW
    _NCHUNK = _BPW // _CHUNK
    mesh = plsc.VectorSubcoreMesh(core_axis_name="c", subcore_axis_name="s")

    @functools.partial(
        pl.kernel,
        mesh=mesh,
        compiler_params=pltpu.CompilerParams(use_tc_tiling_on_sc=False),
        out_type=jax.ShapeDtypeStruct((ntok, DIM), jnp.float32),
        scratch_types=[
            pltpu.VMEM((_NCHUNK, _CHUNK), jnp.int32),
            pltpu.VMEM((_BPW, DIM), jnp.float32),
            pltpu.SemaphoreType.DMA,
            pltpu.SemaphoreType.DMA,
        ],
    )
    def k(table_hbm, idx_hbm, out_hbm, idx_v, rows_v, gsem, wsem):
        wid = lax.axis_index("s") * _NC + lax.axis_index("c")
        pltpu.sync_copy(idx_hbm.at[pl.ds(wid * _NCHUNK, _NCHUNK), :], idx_v)
        gathers = [
            pltpu.async_copy(
                table_hbm.at[idx_v.at[j]],
                rows_v.at[pl.ds(j * _CHUNK, _CHUNK), :],
                gsem,
            )
            for j in range(_NCHUNK)
        ]
        writes = []
        for j in range(_NCHUNK):
            gathers[j].wait()
            writes.append(pltpu.async_copy(
                rows_v.at[pl.ds(j * _CHUNK, _CHUNK), :],
                out_hbm.at[pl.ds(wid * _BPW + j * _CHUNK, _CHUNK), :],
                wsem,
            ))
        for w in writes:
            w.wait()

    return k(table, idx2d)


def kernel(input, embed, bi):
    gate = (jnp.asarray(bi) == 1).astype(jnp.float32).reshape(1, 1)
    table = embed.T
    flat = input.reshape(TOKENS, DIM)
    htok = TOKENS // 2
    inda, ea, da, cb = _tc_stage(flat, embed, gate, GRID // 2, 0, False)
    qa = _sc_gather(table, inda.reshape(htok // _CHUNK, _CHUNK), htok)
    indb, eb, db, qb = _tc_stage(flat, embed, gate, GRID // 2, GRID // 2, True)
    ind_a = inda.reshape(htok)
    keep = ((lax.broadcasted_iota(jnp.int32, (htok, DIM), 1)
             < DIM - POS_DIM)
            == (ind_a < N_EMBED - POS_EMBED)[:, None])
    qa_m = jnp.where(keep, qa, 0.0) * gate[0, 0]
    quantize = jnp.concatenate([qa_m, qb], axis=0).reshape(input.shape)
    diff = jnp.concatenate([da, db], axis=0).reshape(GRID)
    embed_ind = jnp.concatenate([ea, eb], axis=0)
    return quantize, diff, embed_ind, cb


# speedup vs baseline: 1.2611x; 1.0246x over previous
import functools

import jax
import jax.numpy as jnp
from jax import lax
from jax.experimental import pallas as pl
from jax.experimental.pallas import tpu as pltpu
from jax.experimental.pallas import tpu_sc as plsc

DIM = 64
N_EMBED = 1024
POS_DIM = 16
POS_EMBED = 128
TOKENS = 16384
BLOCK = 1024
GRID = TOKENS // BLOCK


def _tc_common(x, emb, gate):
    row = lax.broadcasted_iota(jnp.int32, (DIM, N_EMBED), 0)
    col = lax.broadcasted_iota(jnp.int32, (DIM, N_EMBED), 1)
    mask = ((row < DIM - POS_DIM) == (col < N_EMBED - POS_EMBED)).astype(
        jnp.float32)
    emb_masked = emb * mask
    cb = emb_masked * gate

    xcb = jnp.dot(x, cb, preferred_element_type=jnp.float32)
    cnorm = jnp.sum(cb * cb, axis=0, keepdims=True)
    s = cnorm - 2.0 * xcb
    mins = jnp.min(s, axis=1, keepdims=True)
    onehot = (s <= mins).astype(jnp.float32)
    r2 = lax.broadcasted_iota(jnp.int32, (2, N_EMBED), 0)
    j2 = lax.broadcasted_iota(jnp.int32, (2, N_EMBED), 1)
    w2 = jnp.where(r2 == 0, j2 // 8, j2 % 8).astype(jnp.float32)
    hl = lax.dot_general(w2, onehot, (((1,), (1,)), ((), ())),
                         preferred_element_type=jnp.float32)
    ind = 8.0 * lax.slice(hl, (0, 0), (1, BLOCK)) + lax.slice(
        hl, (1, 0), (2, BLOCK))
    ind = jnp.minimum(ind, jnp.float32(N_EMBED - 1))
    ind_row = ind.astype(jnp.int32)

    fnorm = jnp.sum(x * x, axis=1, keepdims=True)
    diff = ((jnp.sum(fnorm) + jnp.sum(mins)) / jnp.float32(BLOCK * DIM))
    return emb_masked, cb, onehot, ind_row, diff


def _tc_body_a(x_ref, emb_ref, gate_ref, ind_ref, eind_ref, diff_ref, cb_ref):
    b = pl.program_id(0)
    emb_masked, _, _, ind_row, diff = _tc_common(
        x_ref[...], emb_ref[...], gate_ref[0, 0])

    @pl.when(b == 0)
    def _():
        cb_ref[...] = emb_masked

    ind_ref[...] = ind_row.reshape(1, 1, N_EMBED)
    eind_ref[...] = ind_row.reshape(1, 32, 32)
    diff_ref[...] = diff.reshape(1, 1, 1)


def _tc_stage_a(flat, embed, gate):
    grid = GRID // 2
    return pl.pallas_call(
        _tc_body_a,
        grid=(grid,),
        in_specs=[
            pl.BlockSpec((BLOCK, DIM), lambda b: (b, 0)),
            pl.BlockSpec((DIM, N_EMBED), lambda b: (0, 0)),
            pl.BlockSpec((1, 1), lambda b: (0, 0)),
        ],
        out_specs=[
            pl.BlockSpec((1, 1, N_EMBED), lambda b: (b, 0, 0)),
            pl.BlockSpec((1, 32, 32), lambda b: (b, 0, 0)),
            pl.BlockSpec((1, 1, 1), lambda b: (b, 0, 0)),
            pl.BlockSpec((DIM, N_EMBED), lambda b: (0, 0)),
        ],
        out_shape=[
            jax.ShapeDtypeStruct((grid, 1, N_EMBED), jnp.int32),
            jax.ShapeDtypeStruct((GRID, 32, 32), jnp.int32),
            jax.ShapeDtypeStruct((GRID, 1, 1), jnp.float32),
            jax.ShapeDtypeStruct((DIM, N_EMBED), jnp.float32),
        ],
    )(flat, embed, gate)


def _tc_body_b(x_ref, emb_ref, gate_ref, ea_ref, da_ref, eind_ref, diff_ref,
               q_ref):
    del ea_ref, da_ref
    _, cb, onehot, ind_row, diff = _tc_common(
        x_ref[...], emb_ref[...], gate_ref[0, 0])
    eind_ref[...] = ind_row.reshape(1, 32, 32)
    diff_ref[...] = diff.reshape(1, 1, 1)
    cb_hi = cb.astype(jnp.bfloat16).astype(jnp.float32)
    cb_lo = cb - cb_hi
    cb_hl = jnp.concatenate([cb_hi, cb_lo], axis=0)
    nt = (((1,), (1,)), ((), ()))
    q2 = lax.dot_general(onehot, cb_hl, nt,
                         preferred_element_type=jnp.float32)
    q_ref[...] = (lax.slice(q2, (0, 0), (BLOCK, DIM))
                  + lax.slice(q2, (0, DIM), (BLOCK, 2 * DIM)))


def _tc_stage_b(flat, embed, gate, eind_a, diff_a):
    grid = GRID // 2
    base = GRID // 2
    return pl.pallas_call(
        _tc_body_b,
        grid=(grid,),
        in_specs=[
            pl.BlockSpec((BLOCK, DIM), lambda b: (b + base, 0)),
            pl.BlockSpec((DIM, N_EMBED), lambda b: (0, 0)),
            pl.BlockSpec((1, 1), lambda b: (0, 0)),
            pl.BlockSpec(memory_space=pl.ANY),
            pl.BlockSpec(memory_space=pl.ANY),
        ],
        out_specs=[
            pl.BlockSpec((1, 32, 32), lambda b: (b + base, 0, 0)),
            pl.BlockSpec((1, 1, 1), lambda b: (b + base, 0, 0)),
            pl.BlockSpec((BLOCK, DIM), lambda b: (b, 0)),
        ],
        out_shape=[
            jax.ShapeDtypeStruct((GRID, 32, 32), jnp.int32),
            jax.ShapeDtypeStruct((GRID, 1, 1), jnp.float32),
            jax.ShapeDtypeStruct((grid * BLOCK, DIM), jnp.float32),
        ],
        input_output_aliases={3: 0, 4: 1},
    )(flat, embed, gate, eind_a, diff_a)


_CHUNK = 128


def _sc_gather(table, idx2d, ntok):
    info = plsc.get_sparse_core_info()
    _NC, _NS = info.num_cores, info.num_subcores
    _NW = _NC * _NS
    _BPW = ntok // _NW
    _NCHUNK = _BPW // _CHUNK
    mesh = plsc.VectorSubcoreMesh(core_axis_name="c", subcore_axis_name="s")

    @functools.partial(
        pl.kernel,
        mesh=mesh,
        compiler_params=pltpu.CompilerParams(use_tc_tiling_on_sc=False),
        out_type=jax.ShapeDtypeStruct((ntok, DIM), jnp.float32),
        scratch_types=[
            pltpu.VMEM((_NCHUNK, _CHUNK), jnp.int32),
            pltpu.VMEM((_BPW, DIM), jnp.float32),
            pltpu.SemaphoreType.DMA,
            pltpu.SemaphoreType.DMA,
        ],
    )
    def k(table_hbm, idx_hbm, out_hbm, idx_v, rows_v, gsem, wsem):
        wid = lax.axis_index("s") * _NC + lax.axis_index("c")
        pltpu.sync_copy(idx_hbm.at[pl.ds(wid * _NCHUNK, _NCHUNK), :], idx_v)
        gathers = [
            pltpu.async_copy(
                table_hbm.at[idx_v.at[j]],
                rows_v.at[pl.ds(j * _CHUNK, _CHUNK), :],
                gsem,
            )
            for j in range(_NCHUNK)
        ]
        writes = []
        for j in range(_NCHUNK):
            gathers[j].wait()
            writes.append(pltpu.async_copy(
                rows_v.at[pl.ds(j * _CHUNK, _CHUNK), :],
                out_hbm.at[pl.ds(wid * _BPW + j * _CHUNK, _CHUNK), :],
                wsem,
            ))
        for w in writes:
            w.wait()

    return k(table, idx2d)


def kernel(input, embed, bi):
    gate = (jnp.asarray(bi) == 1).astype(jnp.float32).reshape(1, 1)
    table = embed.T
    flat = input.reshape(TOKENS, DIM)
    htok = TOKENS // 2
    inda, ea, da, cb = _tc_stage_a(flat, embed, gate)
    qa = _sc_gather(table, inda.reshape(htok // _CHUNK, _CHUNK), htok)
    embed_ind, diff, qb = _tc_stage_b(flat, embed, gate, ea, da)
    ind_a = inda.reshape(htok)
    keep = ((lax.broadcasted_iota(jnp.int32, (htok, DIM), 1)
             < DIM - POS_DIM)
            == (ind_a < N_EMBED - POS_EMBED)[:, None])
    qa_m = jnp.where(keep, qa, 0.0) * gate[0, 0]
    quantize = jnp.concatenate([qa_m, qb], axis=0).reshape(input.shape)
    return quantize, diff.reshape(GRID), embed_ind, cb


# speedup vs baseline: 1.3313x; 1.0557x over previous
import functools

import jax
import jax.numpy as jnp
from jax import lax
from jax.experimental import pallas as pl
from jax.experimental.pallas import tpu as pltpu
from jax.experimental.pallas import tpu_sc as plsc

DIM = 64
N_EMBED = 1024
POS_DIM = 16
POS_EMBED = 128
TOKENS = 16384
BLOCK = 1024
GRID = TOKENS // BLOCK


def _tc_common(x, emb, gate):
    row = lax.broadcasted_iota(jnp.int32, (DIM, N_EMBED), 0)
    col = lax.broadcasted_iota(jnp.int32, (DIM, N_EMBED), 1)
    mask = ((row < DIM - POS_DIM) == (col < N_EMBED - POS_EMBED)).astype(
        jnp.float32)
    emb_masked = emb * mask
    cb = emb_masked * gate

    xcb = jnp.dot(x, cb, preferred_element_type=jnp.float32)
    cnorm = jnp.sum(cb * cb, axis=0, keepdims=True)
    s = cnorm - 2.0 * xcb
    mins = jnp.min(s, axis=1, keepdims=True)
    onehot = (s <= mins).astype(jnp.float32)
    r2 = lax.broadcasted_iota(jnp.int32, (2, N_EMBED), 0)
    j2 = lax.broadcasted_iota(jnp.int32, (2, N_EMBED), 1)
    w2 = jnp.where(r2 == 0, j2 // 8, j2 % 8).astype(jnp.float32)
    hl = lax.dot_general(w2, onehot, (((1,), (1,)), ((), ())),
                         preferred_element_type=jnp.float32)
    ind = 8.0 * lax.slice(hl, (0, 0), (1, BLOCK)) + lax.slice(
        hl, (1, 0), (2, BLOCK))
    ind = jnp.minimum(ind, jnp.float32(N_EMBED - 1))
    ind_row = ind.astype(jnp.int32)

    fnorm = jnp.sum(x * x, axis=1, keepdims=True)
    diff = ((jnp.sum(fnorm) + jnp.sum(mins)) / jnp.float32(BLOCK * DIM))
    return emb_masked, cb, onehot, ind_row, diff


def _tc_body_a(x_ref, emb_ref, gate_ref, ind_ref, eind_ref, diff_ref, cb_ref):
    b = pl.program_id(0)
    emb_masked, _, _, ind_row, diff = _tc_common(
        x_ref[...], emb_ref[...], gate_ref[0, 0])

    @pl.when(b == 0)
    def _():
        cb_ref[...] = emb_masked

    ind_sc = ind_row + jnp.where(ind_row >= N_EMBED - POS_EMBED, N_EMBED, 0)
    ind_ref[...] = ind_sc.reshape(1, 1, N_EMBED)
    eind_ref[...] = ind_row.reshape(1, 32, 32)
    diff_ref[...] = diff.reshape(1, 1, 1)


def _tc_stage_a(flat, embed, gate):
    grid = GRID // 2
    return pl.pallas_call(
        _tc_body_a,
        grid=(grid,),
        in_specs=[
            pl.BlockSpec((BLOCK, DIM), lambda b: (b, 0)),
            pl.BlockSpec((DIM, N_EMBED), lambda b: (0, 0)),
            pl.BlockSpec((1, 1), lambda b: (0, 0)),
        ],
        out_specs=[
            pl.BlockSpec((1, 1, N_EMBED), lambda b: (b, 0, 0)),
            pl.BlockSpec((1, 32, 32), lambda b: (b, 0, 0)),
            pl.BlockSpec((1, 1, 1), lambda b: (b, 0, 0)),
            pl.BlockSpec((DIM, N_EMBED), lambda b: (0, 0)),
        ],
        out_shape=[
            jax.ShapeDtypeStruct((grid, 1, N_EMBED), jnp.int32),
            jax.ShapeDtypeStruct((GRID, 32, 32), jnp.int32),
            jax.ShapeDtypeStruct((GRID, 1, 1), jnp.float32),
            jax.ShapeDtypeStruct((DIM, N_EMBED), jnp.float32),
        ],
    )(flat, embed, gate)


def _tc_body_b(x_ref, emb_ref, gate_ref, ea_ref, da_ref, eind_ref, diff_ref,
               q_ref):
    del ea_ref, da_ref
    _, cb, onehot, ind_row, diff = _tc_common(
        x_ref[...], emb_ref[...], gate_ref[0, 0])
    eind_ref[...] = ind_row.reshape(1, 32, 32)
    diff_ref[...] = diff.reshape(1, 1, 1)
    cb_hi = cb.astype(jnp.bfloat16).astype(jnp.float32)
    cb_lo = cb - cb_hi
    cb_hl = jnp.concatenate([cb_hi, cb_lo], axis=0)
    nt = (((1,), (1,)), ((), ()))
    q2 = lax.dot_general(onehot, cb_hl, nt,
                         preferred_element_type=jnp.float32)
    q_ref[...] = (lax.slice(q2, (0, 0), (BLOCK, DIM))
                  + lax.slice(q2, (0, DIM), (BLOCK, 2 * DIM)))


def _tc_stage_b(flat, embed, gate, eind_a, diff_a):
    grid = GRID // 2
    base = GRID // 2
    return pl.pallas_call(
        _tc_body_b,
        grid=(grid,),
        in_specs=[
            pl.BlockSpec((BLOCK, DIM), lambda b: (b + base, 0)),
            pl.BlockSpec((DIM, N_EMBED), lambda b: (0, 0)),
            pl.BlockSpec((1, 1), lambda b: (0, 0)),
            pl.BlockSpec(memory_space=pl.ANY),
            pl.BlockSpec(memory_space=pl.ANY),
        ],
        out_specs=[
            pl.BlockSpec((1, 32, 32), lambda b: (b + base, 0, 0)),
            pl.BlockSpec((1, 1, 1), lambda b: (b + base, 0, 0)),
            pl.BlockSpec((BLOCK, DIM), lambda b: (b, 0)),
        ],
        out_shape=[
            jax.ShapeDtypeStruct((GRID, 32, 32), jnp.int32),
            jax.ShapeDtypeStruct((GRID, 1, 1), jnp.float32),
            jax.ShapeDtypeStruct((grid * BLOCK, DIM), jnp.float32),
        ],
        input_output_aliases={3: 0, 4: 1},
    )(flat, embed, gate, eind_a, diff_a)


_CHUNK = 128


def _sc_gather(table, idx2d, ntok):
    info = plsc.get_sparse_core_info()
    _NC, _NS = info.num_cores, info.num_subcores
    _NW = _NC * _NS
    _BPW = ntok // _NW
    _NCHUNK = _BPW // _CHUNK
    mesh = plsc.VectorSubcoreMesh(core_axis_name="c", subcore_axis_name="s")

    @functools.partial(
        pl.kernel,
        mesh=mesh,
        compiler_params=pltpu.CompilerParams(use_tc_tiling_on_sc=False),
        out_type=jax.ShapeDtypeStruct((ntok, DIM), jnp.float32),
        scratch_types=[
            pltpu.VMEM((_NCHUNK, _CHUNK), jnp.int32),
            pltpu.VMEM((_BPW, DIM), jnp.float32),
            pltpu.SemaphoreType.DMA,
            pltpu.SemaphoreType.DMA,
        ],
    )
    def k(table_hbm, idx_hbm, out_hbm, idx_v, rows_v, gsem, wsem):
        wid = lax.axis_index("s") * _NC + lax.axis_index("c")
        pltpu.sync_copy(idx_hbm.at[pl.ds(wid * _NCHUNK, _NCHUNK), :], idx_v)
        gathers = [
            pltpu.async_copy(
                table_hbm.at[idx_v.at[j]],
                rows_v.at[pl.ds(j * _CHUNK, _CHUNK), :],
                gsem,
            )
            for j in range(_NCHUNK)
        ]
        writes = []
        for j in range(_NCHUNK):
            gathers[j].wait()
            writes.append(pltpu.async_copy(
                rows_v.at[pl.ds(j * _CHUNK, _CHUNK), :],
                out_hbm.at[pl.ds(wid * _BPW + j * _CHUNK, _CHUNK), :],
                wsem,
            ))
        for w in writes:
            w.wait()

    return k(table, idx2d)


def kernel(input, embed, bi):
    gate = (jnp.asarray(bi) == 1).astype(jnp.float32).reshape(1, 1)
    et = embed.T * gate[0, 0]
    dmask = (jnp.arange(DIM) < DIM - POS_DIM).astype(jnp.float32)[None, :]
    table = jnp.concatenate([et * dmask, et * (1.0 - dmask)], axis=0)
    flat = input.reshape(TOKENS, DIM)
    htok = TOKENS // 2
    inda, ea, da, cb = _tc_stage_a(flat, embed, gate)
    qa = _sc_gather(table, inda.reshape(htok // _CHUNK, _CHUNK), htok)
    embed_ind, diff, qb = _tc_stage_b(flat, embed, gate, ea, da)
    quantize = jnp.concatenate([qa, qb], axis=0).reshape(input.shape)
    return quantize, diff.reshape(GRID), embed_ind, cb


# speedup vs baseline: 1.3478x; 1.0124x over previous
import functools

import jax
import jax.numpy as jnp
from jax import lax
from jax.experimental import pallas as pl
from jax.experimental.pallas import tpu as pltpu
from jax.experimental.pallas import tpu_sc as plsc

DIM = 64
N_EMBED = 1024
POS_DIM = 16
POS_EMBED = 128
TOKENS = 16384
BLOCK = 1024
GRID = TOKENS // BLOCK


def _tc_common(x, emb, gate):
    row = lax.broadcasted_iota(jnp.int32, (DIM, N_EMBED), 0)
    col = lax.broadcasted_iota(jnp.int32, (DIM, N_EMBED), 1)
    mask = ((row < DIM - POS_DIM) == (col < N_EMBED - POS_EMBED)).astype(
        jnp.float32)
    emb_masked = emb * mask
    cb = emb_masked * gate

    xcb = jnp.dot(x, cb, preferred_element_type=jnp.float32)
    cnorm = jnp.sum(cb * cb, axis=0, keepdims=True)
    s = cnorm - 2.0 * xcb
    mins = jnp.min(s, axis=1, keepdims=True)
    onehot = (s <= mins).astype(jnp.float32)
    r2 = lax.broadcasted_iota(jnp.int32, (2, N_EMBED), 0)
    j2 = lax.broadcasted_iota(jnp.int32, (2, N_EMBED), 1)
    w2 = jnp.where(r2 == 0, j2 // 8, j2 % 8).astype(jnp.float32)
    hl = lax.dot_general(w2, onehot, (((1,), (1,)), ((), ())),
                         preferred_element_type=jnp.float32)
    ind = 8.0 * lax.slice(hl, (0, 0), (1, BLOCK)) + lax.slice(
        hl, (1, 0), (2, BLOCK))
    ind = jnp.minimum(ind, jnp.float32(N_EMBED - 1))
    ind_row = ind.astype(jnp.int32)

    fnorm = jnp.sum(x * x, axis=1, keepdims=True)
    diff = ((jnp.sum(fnorm) + jnp.sum(mins)) / jnp.float32(BLOCK * DIM))
    return emb_masked, cb, onehot, ind_row, diff


def _tc_body_a(x_ref, emb_ref, gate_ref, ind_ref, eind_ref, diff_ref, cb_ref):
    b = pl.program_id(0)
    emb_masked, _, _, ind_row, diff = _tc_common(
        x_ref[...], emb_ref[...], gate_ref[0, 0])

    @pl.when(b == 0)
    def _():
        cb_ref[...] = emb_masked

    ind_sc = ind_row + jnp.where(ind_row >= N_EMBED - POS_EMBED, N_EMBED, 0)
    ind_ref[...] = ind_sc.reshape(1, 1, N_EMBED)
    eind_ref[...] = ind_row.reshape(1, 32, 32)
    diff_ref[...] = diff.reshape(1, 1, 1)


def _tc_stage_a(flat, embed, gate):
    grid = GRID // 2
    return pl.pallas_call(
        _tc_body_a,
        grid=(grid,),
        in_specs=[
            pl.BlockSpec((BLOCK, DIM), lambda b: (b, 0)),
            pl.BlockSpec((DIM, N_EMBED), lambda b: (0, 0)),
            pl.BlockSpec((1, 1), lambda b: (0, 0)),
        ],
        out_specs=[
            pl.BlockSpec((1, 1, N_EMBED), lambda b: (b, 0, 0)),
            pl.BlockSpec((1, 32, 32), lambda b: (b, 0, 0)),
            pl.BlockSpec((1, 1, 1), lambda b: (b, 0, 0)),
            pl.BlockSpec((DIM, N_EMBED), lambda b: (0, 0)),
        ],
        out_shape=[
            jax.ShapeDtypeStruct((grid, 1, N_EMBED), jnp.int32),
            jax.ShapeDtypeStruct((GRID, 32, 32), jnp.int32),
            jax.ShapeDtypeStruct((GRID, 1, 1), jnp.float32),
            jax.ShapeDtypeStruct((DIM, N_EMBED), jnp.float32),
        ],
    )(flat, embed, gate)


def _tc_body_b(x_ref, emb_ref, gate_ref, ea_ref, da_ref, eind_ref, diff_ref,
               q_ref):
    del ea_ref, da_ref
    _, cb, onehot, ind_row, diff = _tc_common(
        x_ref[...], emb_ref[...], gate_ref[0, 0])
    eind_ref[...] = ind_row.reshape(1, 32, 32)
    diff_ref[...] = diff.reshape(1, 1, 1)
    cb_hi = cb.astype(jnp.bfloat16).astype(jnp.float32)
    cb_lo = cb - cb_hi
    cb_hl = jnp.concatenate([cb_hi, cb_lo], axis=0)
    nt = (((1,), (1,)), ((), ()))
    q2 = lax.dot_general(onehot, cb_hl, nt,
                         preferred_element_type=jnp.float32)
    q_ref[...] = (lax.slice(q2, (0, 0), (BLOCK, DIM))
                  + lax.slice(q2, (0, DIM), (BLOCK, 2 * DIM)))


def _tc_stage_b(flat, embed, gate, eind_a, diff_a):
    grid = GRID // 2
    base = GRID // 2
    return pl.pallas_call(
        _tc_body_b,
        grid=(grid,),
        in_specs=[
            pl.BlockSpec((BLOCK, DIM), lambda b: (b + base, 0)),
            pl.BlockSpec((DIM, N_EMBED), lambda b: (0, 0)),
            pl.BlockSpec((1, 1), lambda b: (0, 0)),
            pl.BlockSpec(memory_space=pl.ANY),
            pl.BlockSpec(memory_space=pl.ANY),
        ],
        out_specs=[
            pl.BlockSpec((1, 32, 32), lambda b: (b + base, 0, 0)),
            pl.BlockSpec((1, 1, 1), lambda b: (b + base, 0, 0)),
            pl.BlockSpec((BLOCK, DIM), lambda b: (b, 0)),
        ],
        out_shape=[
            jax.ShapeDtypeStruct((GRID, 32, 32), jnp.int32),
            jax.ShapeDtypeStruct((GRID, 1, 1), jnp.float32),
            jax.ShapeDtypeStruct((grid * BLOCK, DIM), jnp.float32),
        ],
        input_output_aliases={3: 0, 4: 1},
    )(flat, embed, gate, eind_a, diff_a)


_CHUNK = 128


def _sc_gather(table, idx2d, ntok):
    W = table.shape[1]
    info = plsc.get_sparse_core_info()
    _NC, _NS = info.num_cores, info.num_subcores
    _NW = _NC * _NS
    _BPW = ntok // _NW
    _NCHUNK = _BPW // _CHUNK
    mesh = plsc.VectorSubcoreMesh(core_axis_name="c", subcore_axis_name="s")

    @functools.partial(
        pl.kernel,
        mesh=mesh,
        compiler_params=pltpu.CompilerParams(use_tc_tiling_on_sc=False),
        out_type=jax.ShapeDtypeStruct((ntok, W), jnp.float32),
        scratch_types=[
            pltpu.VMEM((_NCHUNK, _CHUNK), jnp.int32),
            pltpu.VMEM((_BPW, W), jnp.float32),
            pltpu.SemaphoreType.DMA,
            pltpu.SemaphoreType.DMA,
        ],
    )
    def k(table_hbm, idx_hbm, out_hbm, idx_v, rows_v, gsem, wsem):
        wid = lax.axis_index("s") * _NC + lax.axis_index("c")
        pltpu.sync_copy(idx_hbm.at[pl.ds(wid * _NCHUNK, _NCHUNK), :], idx_v)
        gathers = [
            pltpu.async_copy(
                table_hbm.at[idx_v.at[j]],
                rows_v.at[pl.ds(j * _CHUNK, _CHUNK), :],
                gsem,
            )
            for j in range(_NCHUNK)
        ]
        writes = []
        for j in range(_NCHUNK):
            gathers[j].wait()
            writes.append(pltpu.async_copy(
                rows_v.at[pl.ds(j * _CHUNK, _CHUNK), :],
                out_hbm.at[pl.ds(wid * _BPW + j * _CHUNK, _CHUNK), :],
                wsem,
            ))
        for w in writes:
            w.wait()

    return k(table, idx2d)


def kernel(input, embed, bi):
    gate = (jnp.asarray(bi) == 1).astype(jnp.float32).reshape(1, 1)
    et = embed.T * gate[0, 0]
    dmask = (jnp.arange(DIM) < DIM - POS_DIM).astype(jnp.float32)[None, :]
    zeros = jnp.zeros((2 * N_EMBED, DIM), jnp.float32)
    table = jnp.concatenate(
        [jnp.concatenate([et * dmask, et * (1.0 - dmask)], axis=0), zeros],
        axis=1)
    flat = input.reshape(TOKENS, DIM)
    htok = TOKENS // 2
    inda, ea, da, cb = _tc_stage_a(flat, embed, gate)
    qa = _sc_gather(table, inda.reshape(htok // _CHUNK, _CHUNK), htok)
    embed_ind, diff, qb = _tc_stage_b(flat, embed, gate, ea, da)
    quantize = jnp.concatenate([qa[:, :DIM], qb], axis=0).reshape(input.shape)
    return quantize, diff.reshape(GRID), embed_ind, cb
